# baseline (speedup 1.0000x reference)
import functools

import jax
import jax.numpy as jnp
from jax import lax
from jax.experimental import pallas as pl
from jax.experimental.pallas import tpu as pltpu
from jax.experimental.pallas import tpu_sc as plsc

BATCH = 16384
D = 64
VOCAB = 100000
NUM_CORES = 2
NUM_SUBCORES = 16
NW = NUM_CORES * NUM_SUBCORES
DPW = D // NW
HALF = BATCH // 2
G = 4096


def _sc_transposed_gather(user_idx, pos_idx, neg_idx, user_t, item_t):
    mesh = plsc.VectorSubcoreMesh(core_axis_name="c", subcore_axis_name="s")

    @functools.partial(
        pl.kernel,
        mesh=mesh,
        out_type=(
            jax.ShapeDtypeStruct((D, BATCH), jnp.float32),
            jax.ShapeDtypeStruct((D, BATCH), jnp.float32),
        ),
        compiler_params=pltpu.CompilerParams(needs_layout_passes=False),
        scratch_types=[
            pltpu.VMEM((VOCAB,), jnp.float32),
            pltpu.VMEM((HALF,), jnp.int32),
            pltpu.VMEM((HALF,), jnp.int32),
            pltpu.VMEM((HALF,), jnp.float32),
            pltpu.SemaphoreType.DMA,
        ],
    )
    def k(uidx_h, pidx_h, nidx_h, ut_h, it_h, out_u, out_d,
          line, idxp, idxn, oa, sem):
        wid = lax.axis_index("s") * NUM_CORES + lax.axis_index("c")

        def utask(t, carry):
            c = wid * DPW + t
            pltpu.async_copy(ut_h.at[c], line, sem).wait()
            for h in range(2):
                hsl = pl.ds(h * HALF, HALF)
                pltpu.sync_copy(uidx_h.at[hsl], idxp)

                @plsc.parallel_loop(0, HALF // 16, unroll=8)
                def ubody(i):
                    sl = pl.ds(i * 16, 16)
                    oa[sl] = plsc.load_gather(line, [idxp[sl]])

                pltpu.sync_copy(oa, out_u.at[c, hsl])
            return carry

        lax.fori_loop(0, DPW, utask, 0)

        def itask(t, carry):
            c = wid * DPW + t
            pltpu.async_copy(it_h.at[c], line, sem).wait()
            for h in range(2):
                hsl = pl.ds(h * HALF, HALF)
                pltpu.sync_copy(pidx_h.at[hsl], idxp)
                pltpu.sync_copy(nidx_h.at[hsl], idxn)

                @plsc.parallel_loop(0, HALF // 16, unroll=8)
                def ibody(i):
                    sl = pl.ds(i * 16, 16)
                    gp = plsc.load_gather(line, [idxp[sl]])
                    gn = plsc.load_gather(line, [idxn[sl]])
                    oa[sl] = gp - gn

                pltpu.sync_copy(oa, out_d.at[c, hsl])
            return carry

        lax.fori_loop(0, DPW, itask, 0)

    return k(user_idx, pos_idx, neg_idx, user_t, item_t)


def _tc_loss(u_t, d_t):
    def body(u_ref, d_ref, o_ref):
        x = jnp.sum(u_ref[...] * d_ref[...], axis=0, keepdims=True)
        t = -x
        sp = jnp.maximum(t, 0.0) + jnp.log(1.0 + jnp.exp(-jnp.abs(t)))
        o_ref[0, 0] = jnp.sum(sp) / BATCH

    out = pl.pallas_call(
        body,
        out_shape=jax.ShapeDtypeStruct((1, 1), jnp.float32),
        out_specs=pl.BlockSpec(memory_space=pltpu.SMEM),
    )(u_t, d_t)
    return out[0, 0]


def kernel(user_idx, pos_idx, neg_idx, user_emb, item_emb):
    user_t = user_emb.T
    item_t = item_emb.T
    u_t, d_t = _sc_transposed_gather(user_idx, pos_idx, neg_idx, user_t, item_t)
    return _tc_loss(u_t, d_t)


# speedup vs baseline: 1.0264x; 1.0264x over previous
import functools

import jax
import jax.numpy as jnp
from jax import lax
from jax.experimental import pallas as pl
from jax.experimental.pallas import tpu as pltpu
from jax.experimental.pallas import tpu_sc as plsc

BATCH = 16384
D = 64
VOCAB = 100000
NUM_CORES = 2
NUM_SUBCORES = 16
NW = NUM_CORES * NUM_SUBCORES
DPW = D // NW
G = 4096


def _sc_transposed_gather(user_idx, pos_idx, neg_idx, user_t, item_t):
    mesh = plsc.VectorSubcoreMesh(core_axis_name="c", subcore_axis_name="s")

    @functools.partial(
        pl.kernel,
        mesh=mesh,
        out_type=(
            jax.ShapeDtypeStruct((D, BATCH), jnp.float32),
            jax.ShapeDtypeStruct((D, BATCH), jnp.float32),
        ),
        compiler_params=pltpu.CompilerParams(needs_layout_passes=False),
        scratch_types=[
            pltpu.VMEM((VOCAB,), jnp.float32),
            pltpu.VMEM((BATCH,), jnp.int32),
            pltpu.VMEM((G,), jnp.int32),
            pltpu.VMEM((G,), jnp.float32),
            pltpu.SemaphoreType.DMA,
            pltpu.SemaphoreType.DMA,
        ],
    )
    def k(uidx_h, pidx_h, nidx_h, ut_h, it_h, out_u, out_d,
          line, idxa, idxn, oa, seml, semi):
        wid = lax.axis_index("s") * NUM_CORES + lax.axis_index("c")

        ah = pltpu.async_copy(uidx_h, idxa, semi)

        def utask(t, carry):
            c = wid * DPW + t
            pltpu.async_copy(ut_h.at[c], line, seml).wait()
            for q in range(BATCH // G):
                qsl = pl.ds(q * G, G)

                @plsc.parallel_loop(0, G // 16, unroll=8)
                def ubody(i, q=q):
                    sl = pl.ds(i * 16, 16)
                    oa[sl] = plsc.load_gather(
                        line, [idxa[pl.ds(q * G + i * 16, 16)]])

                pltpu.sync_copy(oa, out_u.at[c, qsl])
            return carry

        ah.wait()
        lax.fori_loop(0, DPW, utask, 0)

        ah = pltpu.async_copy(pidx_h, idxa, semi)

        def itask(t, carry):
            c = wid * DPW + t
            pltpu.async_copy(it_h.at[c], line, seml).wait()
            for q in range(BATCH // G):
                qsl = pl.ds(q * G, G)
                pltpu.sync_copy(nidx_h.at[qsl], idxn)

                @plsc.parallel_loop(0, G // 16, unroll=8)
                def ibody(i, q=q):
                    sl = pl.ds(i * 16, 16)
                    gp = plsc.load_gather(
                        line, [idxa[pl.ds(q * G + i * 16, 16)]])
                    gn = plsc.load_gather(line, [idxn[sl]])
                    oa[sl] = gp - gn

                pltpu.sync_copy(oa, out_d.at[c, qsl])
            return carry

        ah.wait()
        lax.fori_loop(0, DPW, itask, 0)

    return k(user_idx, pos_idx, neg_idx, user_t, item_t)


def _tc_loss(u_t, d_t):
    def body(u_ref, d_ref, o_ref):
        x = jnp.sum(u_ref[...] * d_ref[...], axis=0, keepdims=True)
        t = -x
        sp = jnp.maximum(t, 0.0) + jnp.log(1.0 + jnp.exp(-jnp.abs(t)))
        o_ref[0, 0] = jnp.sum(sp) / BATCH

    out = pl.pallas_call(
        body,
        out_shape=jax.ShapeDtypeStruct((1, 1), jnp.float32),
        out_specs=pl.BlockSpec(memory_space=pltpu.SMEM),
    )(u_t, d_t)
    return out[0, 0]


def kernel(user_idx, pos_idx, neg_idx, user_emb, item_emb):
    user_t = user_emb.T
    item_t = item_emb.T
    u_t, d_t = _sc_transposed_gather(user_idx, pos_idx, neg_idx, user_t, item_t)
    return _tc_loss(u_t, d_t)


# speedup vs baseline: 1.0363x; 1.0096x over previous
import functools

import jax
import jax.numpy as jnp
from jax import lax
from jax.experimental import pallas as pl
from jax.experimental.pallas import tpu as pltpu
from jax.experimental.pallas import tpu_sc as plsc

BATCH = 16384
D = 64
VOCAB = 100000
NUM_CORES = 2
NUM_SUBCORES = 16
NW = NUM_CORES * NUM_SUBCORES
DPW = D // NW
G = 4096


def _sc_transposed_gather(user_idx, pos_idx, neg_idx, user_t, item_t):
    mesh = plsc.VectorSubcoreMesh(core_axis_name="c", subcore_axis_name="s")

    @functools.partial(
        pl.kernel,
        mesh=mesh,
        out_type=(
            jax.ShapeDtypeStruct((D, BATCH), jnp.float32),
            jax.ShapeDtypeStruct((D, BATCH), jnp.float32),
        ),
        compiler_params=pltpu.CompilerParams(needs_layout_passes=False),
        scratch_types=[
            pltpu.VMEM((VOCAB,), jnp.float32),
            pltpu.VMEM((BATCH,), jnp.int32),
            pltpu.VMEM((G,), jnp.int32),
            pltpu.VMEM((G,), jnp.float32),
            pltpu.SemaphoreType.DMA,
            pltpu.SemaphoreType.DMA,
        ],
    )
    def k(uidx_h, pidx_h, nidx_h, ut_h, it_h, out_u, out_d,
          line, idxa, idxn, oa, seml, semi):
        wid = lax.axis_index("s") * NUM_CORES + lax.axis_index("c")

        def user_phase():
            ah = pltpu.async_copy(uidx_h, idxa, semi)

            def utask(t, carry):
                c = wid * DPW + t
                pltpu.async_copy(ut_h.at[c], line, seml).wait()
                for q in range(BATCH // G):
                    qsl = pl.ds(q * G, G)

                    @plsc.parallel_loop(0, G // 16, unroll=8)
                    def ubody(i, q=q):
                        sl = pl.ds(i * 16, 16)
                        oa[sl] = plsc.load_gather(
                            line, [idxa[pl.ds(q * G + i * 16, 16)]])

                    pltpu.sync_copy(oa, out_u.at[c, qsl])
                return carry

            ah.wait()
            lax.fori_loop(0, DPW, utask, 0)

        def item_phase():
            ah = pltpu.async_copy(pidx_h, idxa, semi)

            def itask(t, carry):
                c = wid * DPW + t
                pltpu.async_copy(it_h.at[c], line, seml).wait()
                for q in range(BATCH // G):
                    qsl = pl.ds(q * G, G)
                    pltpu.sync_copy(nidx_h.at[qsl], idxn)

                    @plsc.parallel_loop(0, G // 16, unroll=8)
                    def ibody(i, q=q):
                        sl = pl.ds(i * 16, 16)
                        gp = plsc.load_gather(
                            line, [idxa[pl.ds(q * G + i * 16, 16)]])
                        gn = plsc.load_gather(line, [idxn[sl]])
                        oa[sl] = gp - gn

                    pltpu.sync_copy(oa, out_d.at[c, qsl])
                return carry

            ah.wait()
            lax.fori_loop(0, DPW, itask, 0)

        @pl.when(wid % 2 == 0)
        def _():
            user_phase()
            item_phase()

        @pl.when(wid % 2 == 1)
        def _():
            item_phase()
            user_phase()

    return k(user_idx, pos_idx, neg_idx, user_t, item_t)


def _tc_loss(u_t, d_t):
    def body(u_ref, d_ref, o_ref):
        x = jnp.sum(u_ref[...] * d_ref[...], axis=0, keepdims=True)
        t = -x
        sp = jnp.maximum(t, 0.0) + jnp.log(1.0 + jnp.exp(-jnp.abs(t)))
        o_ref[0, 0] = jnp.sum(sp) / BATCH

    out = pl.pallas_call(
        body,
        out_shape=jax.ShapeDtypeStruct((1, 1), jnp.float32),
        out_specs=pl.BlockSpec(memory_space=pltpu.SMEM),
    )(u_t, d_t)
    return out[0, 0]


def kernel(user_idx, pos_idx, neg_idx, user_emb, item_emb):
    user_t = user_emb.T
    item_t = item_emb.T
    u_t, d_t = _sc_transposed_gather(user_idx, pos_idx, neg_idx, user_t, item_t)
    return _tc_loss(u_t, d_t)
